# baseline (speedup 1.0000x reference)
import jax
import jax.numpy as jnp
from jax.experimental import pallas as pl

_B = 4
_N = 256
_E1 = (_N * (_N - 1)) // 2


def _edge_index_body(out_ref):
    e = jax.lax.broadcasted_iota(jnp.int32, (2 * _B, _E1), 1)
    ep = (_E1 - 1) - e
    s = jnp.sqrt(ep.astype(jnp.float32) * 8.0 + 1.0)
    t = ((s - 1.0) * 0.5).astype(jnp.int32)
    t = jnp.where((t + 1) * (t + 2) // 2 <= ep, t + 1, t)
    t = jnp.where(t * (t + 1) // 2 > ep, t - 1, t)
    row = (_N - 2) - t
    col = (_N - 1) - (ep - t * (t + 1) // 2)
    r8 = jax.lax.broadcasted_iota(jnp.int32, (2 * _B, _E1), 0)
    val = jnp.where(r8 < _B, row, col) + (r8 & (_B - 1)) * _N
    out_ref[:, :] = val


def kernel(x_topology, x_temporal, W_gnn, b_gnn, W_mean, b_mean, W_var, b_var, W_w, b_w):
    out = pl.pallas_call(
        _edge_index_body,
        out_shape=jax.ShapeDtypeStruct((2 * _B, _E1), jnp.int32),
    )()
    return out.reshape(2, _B * _E1)
